# baseline (speedup 1.0000x reference)
import jax
import jax.numpy as jnp
import numpy as np
from jax import lax
from jax.experimental import pallas as pl
from jax.experimental.pallas import tpu as pltpu
from jax.experimental.pallas import tpu_sc as plsc

N_USERS = 25000
N_ITEMS = 25000
N = N_USERS + N_ITEMS
EMB = 64
HYP = 128
C_LAYERS = 2
SLOPE = 0.1
E = 800000

F2 = EMB // 2
NC = 2
NS = 16
SUB = 128
NSUB = 3
CH = SUB * NSUB
CPS = 132
E_PAD = NS * CPS * CH
NPAD = 50048
ROWS_PER_SUB = NPAD // NS

BLK = 5000
NBLK = N // BLK
NBLK_H = N_USERS // BLK


def _leaky(x):
    return jnp.where(x >= 0, x, SLOPE * x)


def _spmm_body(x2, rows, cols, vals, zeros, out, acc,
               col0, col1, val0, val1, row0, row1, row2, row3, msg0, msg1,
               lsem0, lsem1, gsem0, gsem1, ssem0, ssem1):
    c_ax = lax.axis_index("c")
    s_ax = lax.axis_index("s")

    cols_b = [col0, col1]
    vals_b = [val0, val1]
    rows_b = [row0, row1, row2, row3]
    msgs_b = [msg0, msg1]
    lsems = [lsem0, lsem1]
    gsems = [gsem0, gsem1]
    ssems = [ssem0, ssem1]

    pltpu.sync_copy(zeros.at[pl.ds(s_ax * ROWS_PER_SUB, ROWS_PER_SUB)],
                    acc.at[pl.ds(s_ax * ROWS_PER_SUB, ROWS_PER_SUB)])
    plsc.subcore_barrier()

    def l_descs(c, b, r):
        base = (s_ax * CPS + c) * CH
        ds = [pltpu.make_async_copy(cols.at[pl.ds(base, CH)], cols_b[b], lsems[b]),
              pltpu.make_async_copy(vals.at[pl.ds(base, CH)], vals_b[b], lsems[b])]
        for j in range(NSUB):
            ds.append(pltpu.make_async_copy(
                rows.at[pl.ds(base + j * SUB, SUB)], rows_b[r].at[j], lsems[b]))
        return ds

    def g_descs(c, b):
        return [pltpu.make_async_copy(
                    x2.at[cols_b[b].at[pl.ds(j * SUB, SUB)]],
                    msgs_b[b].at[pl.ds(j * SUB, SUB)], gsems[b])
                for j in range(NSUB)]

    coff = jnp.full((16,), c_ax, dtype=jnp.int32)

    def adjust(b):
        cb = cols_b[b]

        def body(k, carry):
            v = cb[pl.ds(k * 16, 16)]
            cb[pl.ds(k * 16, 16)] = v + v + coff
            return carry
        lax.fori_loop(0, CH // 16, body, None, unroll=4)

    def w_start(c, b, r):
        for j in range(NSUB):
            pltpu.async_copy(msgs_b[b].at[pl.ds(j * SUB, SUB)],
                             acc.at[rows_b[r].at[j]], ssems[b], add=True)

    def w_wait(c, b, r):
        for j in range(NSUB):
            pltpu.make_async_copy(msgs_b[b].at[pl.ds(j * SUB, SUB)],
                                  acc.at[rows_b[r].at[j]], ssems[b]).wait()

    def scale(b):
        mb = msgs_b[b]
        vb = vals_b[b]

        def body(G, carry):
            vv = vb[pl.ds(G * 16, 16)]
            for t in range(16):
                bv = lax.broadcast(vv[t], (16,))
                e = G * 16 + t
                mb[e, pl.ds(0, 16)] = mb[e, pl.ds(0, 16)] * bv
                mb[e, pl.ds(16, 16)] = mb[e, pl.ds(16, 16)] * bv
            return carry
        lax.fori_loop(0, CH // 16, body, None, unroll=2)

    def chunk_step(c, q):
        b = q % 2
        r = q % 4

        @pl.when(c >= 2)
        def _():
            w_wait(c - 2, b, r)
        for d in g_descs(c, b):
            d.start()

        @pl.when(c >= 1)
        def _():
            for d in g_descs(c - 1, 1 - b):
                d.wait()
            scale(1 - b)
            w_start(c - 1, 1 - b, (q + 3) % 4)

        @pl.when(c + 1 < CPS)
        def _():
            nds = l_descs(c + 1, 1 - b, (q + 1) % 4)
            for d in nds:
                d.start()
            for d in nds:
                d.wait()
            adjust(1 - b)

    p = l_descs(0, 0, 0)
    for d in p:
        d.start()
    for d in p:
        d.wait()
    adjust(0)

    def quad(t, carry):
        for q in range(4):
            chunk_step(t * 4 + q, q)
        return carry
    lax.fori_loop(0, CPS // 4, quad, None)

    last = CPS - 1
    for d in g_descs(last, last % 2):
        d.wait()
    scale(last % 2)
    w_start(last, last % 2, last % 4)
    w_wait(last - 1, (last - 1) % 2, (last - 1) % 4)
    w_wait(last, last % 2, last % 4)

    plsc.subcore_barrier()

    pltpu.sync_copy(acc.at[pl.ds(s_ax * ROWS_PER_SUB, ROWS_PER_SUB)],
                    out.at[pl.ds(s_ax * ROWS_PER_SUB, ROWS_PER_SUB),
                           pl.ds(c_ax * F2, F2)])


_spmm = pl.kernel(
    _spmm_body,
    out_type=jax.ShapeDtypeStruct((NPAD, EMB), jnp.float32),
    mesh=plsc.VectorSubcoreMesh(core_axis_name="c", subcore_axis_name="s",
                                num_cores=NC, num_subcores=NS),
    compiler_params=pltpu.CompilerParams(use_tc_tiling_on_sc=False),
    scratch_types=[
        pltpu.VMEM_SHARED((NPAD, F2), jnp.float32),
        pltpu.VMEM((CH,), jnp.int32),
        pltpu.VMEM((CH,), jnp.int32),
        pltpu.VMEM((CH,), jnp.float32),
        pltpu.VMEM((CH,), jnp.float32),
        pltpu.VMEM((NSUB, SUB), jnp.int32),
        pltpu.VMEM((NSUB, SUB), jnp.int32),
        pltpu.VMEM((NSUB, SUB), jnp.int32),
        pltpu.VMEM((NSUB, SUB), jnp.int32),
        pltpu.VMEM((CH, F2), jnp.float32),
        pltpu.VMEM((CH, F2), jnp.float32),
        pltpu.SemaphoreType.DMA,
        pltpu.SemaphoreType.DMA,
        pltpu.SemaphoreType.DMA,
        pltpu.SemaphoreType.DMA,
        pltpu.SemaphoreType.DMA,
        pltpu.SemaphoreType.DMA,
    ],
)


def _gram_body(x0_ref, xk_ref, uH_ref, iH_ref, V_ref, m_ref, acc_ref):
    h = pl.program_id(0)
    j = pl.program_id(1)
    part = lax.dot_general(x0_ref[...], xk_ref[...],
                           (((0,), (0,)), ((), ())),
                           preferred_element_type=jnp.float32)

    @pl.when(j == 0)
    def _():
        acc_ref[...] = part

    @pl.when(j > 0)
    def _():
        acc_ref[...] = acc_ref[...] + part

    @pl.when(j == NBLK_H - 1)
    def _():
        W = jnp.where(h == 0, uH_ref[...], iH_ref[...])
        lat = _leaky(lax.dot_general(W, acc_ref[...],
                                     (((0,), (0,)), ((), ())),
                                     preferred_element_type=jnp.float32))
        for _ in range(C_LAYERS):
            lat = _leaky(jnp.dot(V_ref[...], lat,
                                 preferred_element_type=jnp.float32)) + lat
        m_ref[0] = jnp.dot(W, lat, preferred_element_type=jnp.float32)


def _gram(x0, xk, uHyper, iHyper, V):
    return pl.pallas_call(
        _gram_body,
        grid=(2, NBLK_H),
        in_specs=[
            pl.BlockSpec((BLK, EMB), lambda h, j: (h * NBLK_H + j, 0)),
            pl.BlockSpec((BLK, EMB), lambda h, j: (h * NBLK_H + j, 0)),
            pl.BlockSpec((EMB, HYP), lambda h, j: (0, 0)),
            pl.BlockSpec((EMB, HYP), lambda h, j: (0, 0)),
            pl.BlockSpec((HYP, HYP), lambda h, j: (0, 0)),
        ],
        out_specs=pl.BlockSpec((1, EMB, EMB), lambda h, j: (h, 0, 0)),
        out_shape=jax.ShapeDtypeStruct((2, EMB, EMB), jnp.float32),
        scratch_shapes=[pltpu.VMEM((EMB, EMB), jnp.float32)],
    )(x0, xk, uHyper, iHyper, V)


def _combine0_body(s_ref, x0_ref, m_ref, gnn_ref, hyp_ref, xn_ref):
    g = _leaky(s_ref[...])
    hypb = _leaky(jnp.dot(x0_ref[...], m_ref[0],
                          preferred_element_type=jnp.float32))
    gnn_ref[...] = g
    hyp_ref[...] = hypb
    xn_ref[...] = g + hypb


def _combine0(s, x0, m):
    return pl.pallas_call(
        _combine0_body,
        grid=(NBLK,),
        in_specs=[
            pl.BlockSpec((BLK, EMB), lambda i: (i, 0)),
            pl.BlockSpec((BLK, EMB), lambda i: (i, 0)),
            pl.BlockSpec((1, EMB, EMB), lambda i: (i // NBLK_H, 0, 0)),
        ],
        out_specs=[
            pl.BlockSpec((BLK, EMB), lambda i: (i, 0)),
            pl.BlockSpec((BLK, EMB), lambda i: (i, 0)),
            pl.BlockSpec((BLK, EMB), lambda i: (i, 0)),
        ],
        out_shape=[
            jax.ShapeDtypeStruct((N, EMB), jnp.float32),
            jax.ShapeDtypeStruct((N, EMB), jnp.float32),
            jax.ShapeDtypeStruct((N, EMB), jnp.float32),
        ],
    )(s, x0, m)


def _combine1_body(s_ref, x0_ref, x1_ref, m_ref, gnn_ref, hyp_ref, out_ref):
    g = _leaky(s_ref[...])
    hypb = _leaky(jnp.dot(x0_ref[...], m_ref[0],
                          preferred_element_type=jnp.float32))
    gnn_ref[...] = g
    hyp_ref[...] = hypb
    out_ref[...] = x0_ref[...] + x1_ref[...] + g + hypb


def _combine1(s, x0, x1, m):
    return pl.pallas_call(
        _combine1_body,
        grid=(NBLK,),
        in_specs=[
            pl.BlockSpec((BLK, EMB), lambda i: (i, 0)),
            pl.BlockSpec((BLK, EMB), lambda i: (i, 0)),
            pl.BlockSpec((BLK, EMB), lambda i: (i, 0)),
            pl.BlockSpec((1, EMB, EMB), lambda i: (i // NBLK_H, 0, 0)),
        ],
        out_specs=[
            pl.BlockSpec((BLK, EMB), lambda i: (i, 0)),
            pl.BlockSpec((BLK, EMB), lambda i: (i, 0)),
            pl.BlockSpec((BLK, EMB), lambda i: (i, 0)),
        ],
        out_shape=[
            jax.ShapeDtypeStruct((N, EMB), jnp.float32),
            jax.ShapeDtypeStruct((N, EMB), jnp.float32),
            jax.ShapeDtypeStruct((N, EMB), jnp.float32),
        ],
    )(s, x0, x1, m)


_npe = E_PAD - E
_ar = np.arange(_npe)
_ROWS_TAIL = np.asarray(N + _ar % (NPAD - N), dtype=np.int32)
_COLS_TAIL = np.asarray(_ar % N, dtype=np.int32)
_VALS_TAIL = np.zeros((_npe,), dtype=np.float32)


def kernel(adj_indices, adj_values, uEmbeds, iEmbeds, uHyper, iHyper, V, keepRate):
    x0 = jnp.concatenate([uEmbeds, iEmbeds], axis=0)

    rows = jnp.concatenate([adj_indices[0], jnp.asarray(_ROWS_TAIL)])
    cols = jnp.concatenate([adj_indices[1], jnp.asarray(_COLS_TAIL)])
    vals = jnp.concatenate([adj_values, jnp.asarray(_VALS_TAIL)])
    zeros = jnp.zeros((NPAD, F2), jnp.float32)

    s0 = _spmm(x0.reshape(2 * N, F2), rows, cols, vals, zeros)
    m0 = _gram(x0, x0, uHyper, iHyper, V)
    gnn0, hyp0, x1 = _combine0(s0, x0, m0)

    s1 = _spmm(x1.reshape(2 * N, F2), rows, cols, vals, zeros)
    m1 = _gram(x0, x1, uHyper, iHyper, V)
    gnn1, hyp1, out = _combine1(s1, x0, x1, m1)

    return (out, (gnn0, gnn1), (hyp0, hyp1))


# speedup vs baseline: 1.8847x; 1.8847x over previous
import jax
import jax.numpy as jnp
import numpy as np
from jax import lax
from jax.experimental import pallas as pl
from jax.experimental.pallas import tpu as pltpu
from jax.experimental.pallas import tpu_sc as plsc

N_USERS = 25000
N_ITEMS = 25000
N = N_USERS + N_ITEMS
EMB = 64
HYP = 128
C_LAYERS = 2
SLOPE = 0.1
E = 800000

F2 = EMB // 2
NC = 2
NS = 16
SUB = 128
NSUB = 3
CH = SUB * NSUB
CPS = 132
E_PAD = NS * CPS * CH
NPAD = 50048
ROWS_PER_SUB = NPAD // NS

BLK = 5000
NBLK = N // BLK
NBLK_H = N_USERS // BLK


def _leaky(x):
    return jnp.where(x >= 0, x, SLOPE * x)


def _spmm_body(x2, rows, cols, vals, zeros, out, acc,
               col0, col1, val0, val1, row0, row1, row2, row3, msg0, msg1,
               lsem0, lsem1, gsem0, gsem1, ssem0, ssem1):
    c_ax = lax.axis_index("c")
    s_ax = lax.axis_index("s")

    cols_b = [col0, col1]
    vals_b = [val0, val1]
    rows_b = [row0, row1, row2, row3]
    msgs_b = [msg0, msg1]
    lsems = [lsem0, lsem1]
    gsems = [gsem0, gsem1]
    ssems = [ssem0, ssem1]

    pltpu.sync_copy(zeros.at[pl.ds(s_ax * ROWS_PER_SUB, ROWS_PER_SUB)],
                    acc.at[pl.ds(s_ax * ROWS_PER_SUB, ROWS_PER_SUB)])
    plsc.subcore_barrier()

    def l_descs(c, b, r):
        base = (s_ax * CPS + c) * CH
        ds = [pltpu.make_async_copy(cols.at[pl.ds(base, CH)], cols_b[b], lsems[b]),
              pltpu.make_async_copy(vals.at[pl.ds(base, CH)], vals_b[b], lsems[b])]
        for j in range(NSUB):
            ds.append(pltpu.make_async_copy(
                rows.at[pl.ds(base + j * SUB, SUB)], rows_b[r].at[j], lsems[b]))
        return ds

    def g_descs(c, b):
        return [pltpu.make_async_copy(
                    x2.at[cols_b[b].at[pl.ds(j * SUB, SUB)]],
                    msgs_b[b].at[pl.ds(j * SUB, SUB)], gsems[b])
                for j in range(NSUB)]

    coff = jnp.full((16,), c_ax, dtype=jnp.int32)

    def adjust(b):
        cb = cols_b[b]

        def body(k, carry):
            v = cb[pl.ds(k * 16, 16)]
            cb[pl.ds(k * 16, 16)] = v + v + coff
            return carry
        lax.fori_loop(0, CH // 16, body, None, unroll=4)

    def w_start(c, b, r):
        for j in range(NSUB):
            pltpu.async_copy(msgs_b[b].at[pl.ds(j * SUB, SUB)],
                             acc.at[rows_b[r].at[j]], ssems[b], add=True)

    def w_wait(c, b, r):
        for j in range(NSUB):
            pltpu.make_async_copy(msgs_b[b].at[pl.ds(j * SUB, SUB)],
                                  acc.at[rows_b[r].at[j]], ssems[b]).wait()

    def scale(b):
        mb = msgs_b[b]
        vb = vals_b[b]

        def body(G, carry):
            vv = vb[pl.ds(G * 16, 16)]
            for t in range(16):
                bv = lax.broadcast(vv[t], (16,))
                e = G * 16 + t
                mb[e, pl.ds(0, 16)] = mb[e, pl.ds(0, 16)] * bv
                mb[e, pl.ds(16, 16)] = mb[e, pl.ds(16, 16)] * bv
            return carry
        lax.fori_loop(0, CH // 16, body, None)

    def chunk_step(c, q):
        b = q % 2
        r = q % 4

        @pl.when(c >= 2)
        def _():
            w_wait(c - 2, b, r)
        for d in g_descs(c, b):
            d.start()

        @pl.when(c >= 1)
        def _():
            for d in g_descs(c - 1, 1 - b):
                d.wait()
            scale(1 - b)
            w_start(c - 1, 1 - b, (q + 3) % 4)

        @pl.when(c + 1 < CPS)
        def _():
            nds = l_descs(c + 1, 1 - b, (q + 1) % 4)
            for d in nds:
                d.start()
            for d in nds:
                d.wait()
            adjust(1 - b)

    p = l_descs(0, 0, 0)
    for d in p:
        d.start()
    for d in p:
        d.wait()
    adjust(0)

    def quad(t, carry):
        for q in range(4):
            chunk_step(t * 4 + q, q)
        return carry
    lax.fori_loop(0, CPS // 4, quad, None)

    last = CPS - 1
    for d in g_descs(last, last % 2):
        d.wait()
    scale(last % 2)
    w_start(last, last % 2, last % 4)
    w_wait(last - 1, (last - 1) % 2, (last - 1) % 4)
    w_wait(last, last % 2, last % 4)

    plsc.subcore_barrier()

    pltpu.sync_copy(acc.at[pl.ds(s_ax * ROWS_PER_SUB, ROWS_PER_SUB)],
                    out.at[pl.ds(s_ax * ROWS_PER_SUB, ROWS_PER_SUB),
                           pl.ds(c_ax * F2, F2)])


_spmm = pl.kernel(
    _spmm_body,
    out_type=jax.ShapeDtypeStruct((NPAD, EMB), jnp.float32),
    mesh=plsc.VectorSubcoreMesh(core_axis_name="c", subcore_axis_name="s",
                                num_cores=NC, num_subcores=NS),
    compiler_params=pltpu.CompilerParams(use_tc_tiling_on_sc=False),
    scratch_types=[
        pltpu.VMEM_SHARED((NPAD, F2), jnp.float32),
        pltpu.VMEM((CH,), jnp.int32),
        pltpu.VMEM((CH,), jnp.int32),
        pltpu.VMEM((CH,), jnp.float32),
        pltpu.VMEM((CH,), jnp.float32),
        pltpu.VMEM((NSUB, SUB), jnp.int32),
        pltpu.VMEM((NSUB, SUB), jnp.int32),
        pltpu.VMEM((NSUB, SUB), jnp.int32),
        pltpu.VMEM((NSUB, SUB), jnp.int32),
        pltpu.VMEM((CH, F2), jnp.float32),
        pltpu.VMEM((CH, F2), jnp.float32),
        pltpu.SemaphoreType.DMA,
        pltpu.SemaphoreType.DMA,
        pltpu.SemaphoreType.DMA,
        pltpu.SemaphoreType.DMA,
        pltpu.SemaphoreType.DMA,
        pltpu.SemaphoreType.DMA,
    ],
)


def _gram_body(x0_ref, xk_ref, uH_ref, iH_ref, V_ref, m_ref, acc_ref):
    h = pl.program_id(0)
    j = pl.program_id(1)
    part = lax.dot_general(x0_ref[...], xk_ref[...],
                           (((0,), (0,)), ((), ())),
                           preferred_element_type=jnp.float32)

    @pl.when(j == 0)
    def _():
        acc_ref[...] = part

    @pl.when(j > 0)
    def _():
        acc_ref[...] = acc_ref[...] + part

    @pl.when(j == NBLK_H - 1)
    def _():
        W = jnp.where(h == 0, uH_ref[...], iH_ref[...])
        lat = _leaky(lax.dot_general(W, acc_ref[...],
                                     (((0,), (0,)), ((), ())),
                                     preferred_element_type=jnp.float32))
        for _ in range(C_LAYERS):
            lat = _leaky(jnp.dot(V_ref[...], lat,
                                 preferred_element_type=jnp.float32)) + lat
        m_ref[0] = jnp.dot(W, lat, preferred_element_type=jnp.float32)


def _gram(x0, xk, uHyper, iHyper, V):
    return pl.pallas_call(
        _gram_body,
        grid=(2, NBLK_H),
        in_specs=[
            pl.BlockSpec((BLK, EMB), lambda h, j: (h * NBLK_H + j, 0)),
            pl.BlockSpec((BLK, EMB), lambda h, j: (h * NBLK_H + j, 0)),
            pl.BlockSpec((EMB, HYP), lambda h, j: (0, 0)),
            pl.BlockSpec((EMB, HYP), lambda h, j: (0, 0)),
            pl.BlockSpec((HYP, HYP), lambda h, j: (0, 0)),
        ],
        out_specs=pl.BlockSpec((1, EMB, EMB), lambda h, j: (h, 0, 0)),
        out_shape=jax.ShapeDtypeStruct((2, EMB, EMB), jnp.float32),
        scratch_shapes=[pltpu.VMEM((EMB, EMB), jnp.float32)],
    )(x0, xk, uHyper, iHyper, V)


def _combine0_body(s_ref, x0_ref, m_ref, gnn_ref, hyp_ref, xn_ref):
    g = _leaky(s_ref[...])
    hypb = _leaky(jnp.dot(x0_ref[...], m_ref[0],
                          preferred_element_type=jnp.float32))
    gnn_ref[...] = g
    hyp_ref[...] = hypb
    xn_ref[...] = g + hypb


def _combine0(s, x0, m):
    return pl.pallas_call(
        _combine0_body,
        grid=(NBLK,),
        in_specs=[
            pl.BlockSpec((BLK, EMB), lambda i: (i, 0)),
            pl.BlockSpec((BLK, EMB), lambda i: (i, 0)),
            pl.BlockSpec((1, EMB, EMB), lambda i: (i // NBLK_H, 0, 0)),
        ],
        out_specs=[
            pl.BlockSpec((BLK, EMB), lambda i: (i, 0)),
            pl.BlockSpec((BLK, EMB), lambda i: (i, 0)),
            pl.BlockSpec((BLK, EMB), lambda i: (i, 0)),
        ],
        out_shape=[
            jax.ShapeDtypeStruct((N, EMB), jnp.float32),
            jax.ShapeDtypeStruct((N, EMB), jnp.float32),
            jax.ShapeDtypeStruct((N, EMB), jnp.float32),
        ],
    )(s, x0, m)


def _combine1_body(s_ref, x0_ref, x1_ref, m_ref, gnn_ref, hyp_ref, out_ref):
    g = _leaky(s_ref[...])
    hypb = _leaky(jnp.dot(x0_ref[...], m_ref[0],
                          preferred_element_type=jnp.float32))
    gnn_ref[...] = g
    hyp_ref[...] = hypb
    out_ref[...] = x0_ref[...] + x1_ref[...] + g + hypb


def _combine1(s, x0, x1, m):
    return pl.pallas_call(
        _combine1_body,
        grid=(NBLK,),
        in_specs=[
            pl.BlockSpec((BLK, EMB), lambda i: (i, 0)),
            pl.BlockSpec((BLK, EMB), lambda i: (i, 0)),
            pl.BlockSpec((BLK, EMB), lambda i: (i, 0)),
            pl.BlockSpec((1, EMB, EMB), lambda i: (i // NBLK_H, 0, 0)),
        ],
        out_specs=[
            pl.BlockSpec((BLK, EMB), lambda i: (i, 0)),
            pl.BlockSpec((BLK, EMB), lambda i: (i, 0)),
            pl.BlockSpec((BLK, EMB), lambda i: (i, 0)),
        ],
        out_shape=[
            jax.ShapeDtypeStruct((N, EMB), jnp.float32),
            jax.ShapeDtypeStruct((N, EMB), jnp.float32),
            jax.ShapeDtypeStruct((N, EMB), jnp.float32),
        ],
    )(s, x0, x1, m)


_npe = E_PAD - E
_ar = np.arange(_npe)
_ROWS_TAIL = np.asarray(N + _ar % (NPAD - N), dtype=np.int32)
_COLS_TAIL = np.asarray(_ar % N, dtype=np.int32)
_VALS_TAIL = np.zeros((_npe,), dtype=np.float32)


def kernel(adj_indices, adj_values, uEmbeds, iEmbeds, uHyper, iHyper, V, keepRate):
    x0 = jnp.concatenate([uEmbeds, iEmbeds], axis=0)

    rows = jnp.concatenate([adj_indices[0], jnp.asarray(_ROWS_TAIL)])
    cols = jnp.concatenate([adj_indices[1], jnp.asarray(_COLS_TAIL)])
    vals = jnp.concatenate([adj_values, jnp.asarray(_VALS_TAIL)])
    zeros = jnp.zeros((NPAD, F2), jnp.float32)

    s0 = _spmm(x0.reshape(2 * N, F2), rows, cols, vals, zeros)
    m0 = _gram(x0, x0, uHyper, iHyper, V)
    gnn0, hyp0, x1 = _combine0(s0, x0, m0)

    s1 = _spmm(x1.reshape(2 * N, F2), rows, cols, vals, zeros)
    m1 = _gram(x0, x1, uHyper, iHyper, V)
    gnn1, hyp1, out = _combine1(s1, x0, x1, m1)

    return (out, (gnn0, gnn1), (hyp0, hyp1))
